# baseline (speedup 1.0000x reference)
import functools

import jax
import jax.numpy as jnp
from jax import lax
from jax.experimental import pallas as pl
from jax.experimental.pallas import tpu as pltpu
from jax.experimental.pallas import tpu_sc as plsc

BATCH = 16384
EMBED = 16
NUM_CORES = 2
NUM_SUBCORES = 16
NUM_WORKERS = NUM_CORES * NUM_SUBCORES
BPW = BATCH // NUM_WORKERS
CHUNK = 128
NCHUNK = BPW // CHUNK


def _sc_gather_dot(uidx2d, pidx2d, user_emb, ub_flat, places_emb, pb_flat):
    mesh = plsc.VectorSubcoreMesh(core_axis_name="c", subcore_axis_name="s")

    @functools.partial(
        pl.kernel,
        mesh=mesh,
        compiler_params=pltpu.CompilerParams(use_tc_tiling_on_sc=False),
        out_type=[
            jax.ShapeDtypeStruct((NUM_WORKERS, EMBED), jnp.float32),
            jax.ShapeDtypeStruct((BATCH,), jnp.float32),
        ],
        scratch_types=[
            pltpu.VMEM((NCHUNK, CHUNK), jnp.int32),
            pltpu.VMEM((NCHUNK, CHUNK), jnp.int32),
            pltpu.VMEM((BPW, EMBED), jnp.float32),
            pltpu.VMEM((BPW, EMBED), jnp.float32),
            pltpu.VMEM((BPW,), jnp.float32),
            pltpu.VMEM((BPW,), jnp.float32),
            pltpu.VMEM((EMBED,), jnp.float32),
            pltpu.SemaphoreType.DMA,
        ],
    )
    def k(uidx_hbm, pidx_hbm, uemb_hbm, ub_hbm, pemb_hbm, pb_hbm,
          part_out, bias_out,
          uidx_v, pidx_v, urows_v, prows_v, ub_v, pb_v, acc_v, sem):
        wid = lax.axis_index("s") * NUM_CORES + lax.axis_index("c")
        base = wid * BPW

        pltpu.sync_copy(uidx_hbm.at[pl.ds(wid * NCHUNK, NCHUNK)], uidx_v)
        pltpu.sync_copy(pidx_hbm.at[pl.ds(wid * NCHUNK, NCHUNK)], pidx_v)

        copies = []
        for j in range(NCHUNK):
            dst = pl.ds(j * CHUNK, CHUNK)
            copies.append(pltpu.async_copy(
                uemb_hbm.at[uidx_v.at[j]], urows_v.at[dst], sem))
            copies.append(pltpu.async_copy(
                pemb_hbm.at[pidx_v.at[j]], prows_v.at[dst], sem))
            copies.append(pltpu.async_copy(
                ub_hbm.at[uidx_v.at[j]], ub_v.at[dst], sem))
            copies.append(pltpu.async_copy(
                pb_hbm.at[pidx_v.at[j]], pb_v.at[dst], sem))
        for c in copies:
            c.wait()

        zero = jnp.zeros((EMBED,), jnp.float32)

        def body(i, accs):
            a0, a1, a2, a3 = accs
            b = i * 8
            a0 = a0 + urows_v[b + 0] * prows_v[b + 0]
            a1 = a1 + urows_v[b + 1] * prows_v[b + 1]
            a2 = a2 + urows_v[b + 2] * prows_v[b + 2]
            a3 = a3 + urows_v[b + 3] * prows_v[b + 3]
            a0 = a0 + urows_v[b + 4] * prows_v[b + 4]
            a1 = a1 + urows_v[b + 5] * prows_v[b + 5]
            a2 = a2 + urows_v[b + 6] * prows_v[b + 6]
            a3 = a3 + urows_v[b + 7] * prows_v[b + 7]
            return (a0, a1, a2, a3)

        a0, a1, a2, a3 = lax.fori_loop(0, BPW // 8, body,
                                       (zero, zero, zero, zero))
        acc_v[...] = (a0 + a1) + (a2 + a3)
        pltpu.sync_copy(acc_v, part_out.at[wid])

        for t in range(BPW // EMBED):
            sl = pl.ds(t * EMBED, EMBED)
            ub_v[sl] = ub_v[sl] + pb_v[sl]
        pltpu.sync_copy(ub_v, bias_out.at[pl.ds(base, BPW)])

    return k(uidx2d, pidx2d, user_emb, ub_flat, places_emb, pb_flat)


def _tc_finish(part_ref, bias_ref, out_ref):
    s = jnp.sum(part_ref[...])
    out_ref[...] = jax.nn.sigmoid(bias_ref[...] + s)


def kernel(inputs, user_embedding, user_bias, places_embedding, places_bias):
    uidx2d = inputs[:, 0].reshape(NUM_WORKERS * NCHUNK, CHUNK)
    pidx2d = inputs[:, 1].reshape(NUM_WORKERS * NCHUNK, CHUNK)
    partials, bias_sum = _sc_gather_dot(
        uidx2d, pidx2d,
        user_embedding, user_bias[:, 0],
        places_embedding, places_bias[:, 0])
    out2d = pl.pallas_call(
        _tc_finish,
        out_shape=jax.ShapeDtypeStruct((128, 128), jnp.float32),
    )(partials, bias_sum.reshape(128, 128))
    return out2d.reshape(BATCH, 1)
